# baseline (speedup 1.0000x reference)
import dataclasses

import jax
import jax.numpy as jnp
from jax import lax
from jax.experimental import pallas as pl
from jax.experimental.pallas import tpu as pltpu
from jax.experimental.pallas import tpu_sc as plsc

D_MODEL = 768
NUM_EXPERTS = 8
HIDDEN = 1536
SEQ = 2048
NUM_ASSIGN = 2 * SEQ
FFN_BLOCK = 128
NUM_FFN_BLOCKS = NUM_ASSIGN // FFN_BLOCK + NUM_EXPERTS - 1
NUM_SLOTS = NUM_FFN_BLOCKS * FFN_BLOCK
CUM_GROUP = 128
NUM_GROUPS = SEQ // CUM_GROUP

SC_CORES = 2
SC_SUBCORES = 16
SC_WORKERS = SC_CORES * SC_SUBCORES
CHUNK = SEQ // SC_WORKERS


def _route_kernel(x_ref, wg_ref, bg_ref,
                  p0_ref, p1_ref, w0_ref, w1_ref, be_ref):
    xb = x_ref[...]
    x16 = xb.astype(jnp.bfloat16)
    logits = (
        jnp.dot(x16, wg_ref[...], preferred_element_type=jnp.float32)
        + bg_ref[...]
    )
    lane = lax.broadcasted_iota(jnp.int32, logits.shape, 1)
    m1 = jnp.max(logits, axis=1, keepdims=True)
    am1 = jnp.min(jnp.where(logits == m1, lane, NUM_EXPERTS), axis=1,
                  keepdims=True)
    masked = jnp.where(lane == am1, -jnp.inf, logits)
    m2 = jnp.max(masked, axis=1, keepdims=True)
    am2 = jnp.min(jnp.where(masked == m2, lane, NUM_EXPERTS), axis=1,
                  keepdims=True)
    prob1 = 1.0 / (1.0 + jnp.exp(m2 - m1))
    prob2 = 1.0 - prob1
    w0_ref[...] = prob1
    w1_ref[...] = prob2

    a1 = (lane == am1).astype(jnp.float32)
    a2 = (lane == am2).astype(jnp.float32)

    rsub = lax.broadcasted_iota(jnp.int32, (CUM_GROUP, CUM_GROUP), 0)
    csub = lax.broadcasted_iota(jnp.int32, (CUM_GROUP, CUM_GROUP), 1)
    ltri = (csub < rsub).astype(jnp.bfloat16)

    def grouped_ex_cumsum(a):
        a16 = a.astype(jnp.bfloat16)
        totals = jnp.concatenate(
            [jnp.sum(a[g * CUM_GROUP:(g + 1) * CUM_GROUP], axis=0,
                     keepdims=True) for g in range(NUM_GROUPS)], axis=0)
        rg = lax.broadcasted_iota(jnp.int32, (NUM_GROUPS, NUM_GROUPS), 0)
        cg = lax.broadcasted_iota(jnp.int32, (NUM_GROUPS, NUM_GROUPS), 1)
        lg = (cg < rg).astype(jnp.bfloat16)
        gpref = jnp.dot(lg, totals.astype(jnp.bfloat16),
                        preferred_element_type=jnp.float32)
        pieces = []
        for g in range(NUM_GROUPS):
            loc = jnp.dot(ltri, a16[g * CUM_GROUP:(g + 1) * CUM_GROUP],
                          preferred_element_type=jnp.float32)
            pieces.append(loc + gpref[g:g + 1])
        return jnp.concatenate(pieces, axis=0), jnp.sum(totals, axis=0,
                                                        keepdims=True)

    r1, cnt1 = grouped_ex_cumsum(a1)
    r2, cnt2 = grouped_ex_cumsum(a2)
    r2 = r2 + cnt1
    counts = cnt1 + cnt2

    nblk = jnp.floor((counts + (FFN_BLOCK - 1)) * (1.0 / FFN_BLOCK))
    ru8 = lax.broadcasted_iota(jnp.int32, (NUM_EXPERTS, NUM_EXPERTS), 0)
    cu8 = lax.broadcasted_iota(jnp.int32, (NUM_EXPERTS, NUM_EXPERTS), 1)
    utri8 = (ru8 < cu8).astype(jnp.bfloat16)
    bstart = jnp.dot(nblk.astype(jnp.bfloat16), utri8,
                     preferred_element_type=jnp.float32)
    start = FFN_BLOCK * bstart

    p0 = jnp.sum(a1 * (r1 + start), axis=1, keepdims=True)
    p1 = jnp.sum(a2 * (r2 + start), axis=1, keepdims=True)
    p0_ref[...] = p0.astype(jnp.int32)
    p1_ref[...] = p1.astype(jnp.int32)

    bi = lax.broadcasted_iota(jnp.int32, (40, NUM_EXPERTS), 0).astype(
        jnp.float32)
    be = jnp.sum((bi >= bstart).astype(jnp.float32), axis=1,
                 keepdims=True) - 1.0
    be = jnp.clip(be, 0.0, NUM_EXPERTS - 1.0)
    be_ref[...] = be.astype(jnp.int32)


def _route(xf, wgh, bg2):
    full = lambda i: (0, 0)
    return pl.pallas_call(
        _route_kernel,
        grid=(1,),
        in_specs=[
            pl.BlockSpec((SEQ, D_MODEL), full),
            pl.BlockSpec((D_MODEL, NUM_EXPERTS), full),
            pl.BlockSpec((1, NUM_EXPERTS), full),
        ],
        out_specs=[
            pl.BlockSpec((SEQ, 1), full),
            pl.BlockSpec((SEQ, 1), full),
            pl.BlockSpec((SEQ, 1), full),
            pl.BlockSpec((SEQ, 1), full),
            pl.BlockSpec((40, 1), full),
        ],
        out_shape=[
            jax.ShapeDtypeStruct((SEQ, 1), jnp.int32),
            jax.ShapeDtypeStruct((SEQ, 1), jnp.int32),
            jax.ShapeDtypeStruct((SEQ, 1), jnp.float32),
            jax.ShapeDtypeStruct((SEQ, 1), jnp.float32),
            jax.ShapeDtypeStruct((40, 1), jnp.int32),
        ],
    )(xf, wgh, bg2)


def _dispatch(xb, p0, p1):
    mesh = plsc.VectorSubcoreMesh(core_axis_name="c", subcore_axis_name="s")

    @pl.kernel(
        out_type=jax.ShapeDtypeStruct((NUM_SLOTS, D_MODEL), jnp.float32),
        mesh=mesh,
        scratch_types=[
            pltpu.VMEM((CHUNK, D_MODEL), jnp.float32),
            pltpu.VMEM((CHUNK,), jnp.int32),
            pltpu.SemaphoreType.DMA,
        ],
    )
    def k(x_hbm, p0_hbm, p1_hbm, xs_hbm, rows_v, idx_v, sem):
        wid = lax.axis_index("s") * SC_CORES + lax.axis_index("c")
        base = wid * CHUNK
        pltpu.sync_copy(x_hbm.at[pl.ds(base, CHUNK)], rows_v)
        pltpu.sync_copy(p0_hbm.at[pl.ds(base, CHUNK)], idx_v)
        pltpu.async_copy(rows_v, xs_hbm.at[idx_v], sem).wait()
        pltpu.sync_copy(p1_hbm.at[pl.ds(base, CHUNK)], idx_v)
        pltpu.async_copy(rows_v, xs_hbm.at[idx_v], sem).wait()

    return k(xb, p0, p1)


def _ffn_kernel(be_ref, xs_ref, w1_ref, b1_ref, w2_ref, b2_ref, o_ref):
    h = jnp.dot(xs_ref[...].astype(jnp.bfloat16), w1_ref[0],
                preferred_element_type=jnp.float32)
    h = h + b1_ref[0]
    h = h * jax.nn.sigmoid(h)
    out = jnp.dot(h.astype(jnp.bfloat16), w2_ref[0],
                  preferred_element_type=jnp.float32)
    o_ref[...] = out + b2_ref[0]


def _ffn(be, xs, w1, b1r, w2, b2r):
    grid_spec = pltpu.PrefetchScalarGridSpec(
        num_scalar_prefetch=1,
        grid=(NUM_FFN_BLOCKS,),
        in_specs=[
            pl.BlockSpec((FFN_BLOCK, D_MODEL), lambda i, be: (i, 0)),
            pl.BlockSpec((1, D_MODEL, HIDDEN), lambda i, be: (be[i], 0, 0)),
            pl.BlockSpec((1, 1, HIDDEN), lambda i, be: (be[i], 0, 0)),
            pl.BlockSpec((1, HIDDEN, D_MODEL), lambda i, be: (be[i], 0, 0)),
            pl.BlockSpec((1, 1, D_MODEL), lambda i, be: (be[i], 0, 0)),
        ],
        out_specs=pl.BlockSpec((FFN_BLOCK, D_MODEL), lambda i, be: (i, 0)),
    )
    return pl.pallas_call(
        _ffn_kernel,
        grid_spec=grid_spec,
        out_shape=jax.ShapeDtypeStruct((NUM_SLOTS, D_MODEL), jnp.float32),
        compiler_params=pltpu.CompilerParams(
            dimension_semantics=("arbitrary",),
        ),
    )(be, xs, w1, b1r, w2, b2r)


def _combine(outs, p0, p1, w0, w1):
    mesh = plsc.VectorSubcoreMesh(core_axis_name="c", subcore_axis_name="s")
    cp = pltpu.CompilerParams()
    if "needs_layout_passes" in pltpu.CompilerParams.__dataclass_fields__:
        cp = dataclasses.replace(cp, needs_layout_passes=False)

    @pl.kernel(
        out_type=jax.ShapeDtypeStruct((SEQ, D_MODEL), jnp.float32),
        mesh=mesh,
        compiler_params=cp,
        scratch_types=[
            pltpu.VMEM((CHUNK, D_MODEL), jnp.float32),
            pltpu.VMEM((CHUNK, D_MODEL), jnp.float32),
            pltpu.VMEM((CHUNK,), jnp.int32),
            pltpu.VMEM((CHUNK,), jnp.float32),
            pltpu.VMEM((CHUNK,), jnp.float32),
            pltpu.SemaphoreType.DMA,
        ],
    )
    def k(outs_hbm, p0_hbm, p1_hbm, w0_hbm, w1_hbm, y_hbm,
          r0_v, r1_v, idx_v, w0_v, w1_v, sem):
        wid = lax.axis_index("s") * SC_CORES + lax.axis_index("c")
        base = wid * CHUNK
        pltpu.sync_copy(p0_hbm.at[pl.ds(base, CHUNK)], idx_v)
        pltpu.async_copy(outs_hbm.at[idx_v], r0_v, sem).wait()
        pltpu.sync_copy(p1_hbm.at[pl.ds(base, CHUNK)], idx_v)
        pltpu.async_copy(outs_hbm.at[idx_v], r1_v, sem).wait()
        pltpu.sync_copy(w0_hbm.at[pl.ds(base, CHUNK)], w0_v)
        pltpu.sync_copy(w1_hbm.at[pl.ds(base, CHUNK)], w1_v)

        @pl.loop(0, CHUNK)
        def _(i):
            isplat = jnp.full((16,), i, jnp.int32)
            wa = plsc.load_gather(w0_v, [isplat])
            wb = plsc.load_gather(w1_v, [isplat])

            @pl.loop(0, D_MODEL, step=16)
            def _(j):
                r0_v.at[i, pl.ds(j, 16)][...] = (
                    wa * r0_v.at[i, pl.ds(j, 16)][...]
                    + wb * r1_v.at[i, pl.ds(j, 16)][...]
                )

        pltpu.sync_copy(r0_v, y_hbm.at[pl.ds(base, CHUNK)])

    return k(outs, p0, p1, w0, w1)


def kernel(x, Wg, bg, W1, b1, W2, b2):
    b, s, d = x.shape
    xf = x.reshape(s, d)
    wgh = Wg.astype(jnp.bfloat16)
    bg2 = bg.reshape(1, NUM_EXPERTS)
    w1 = W1.astype(jnp.bfloat16)
    w2 = W2.astype(jnp.bfloat16)
    b1r = b1.reshape(NUM_EXPERTS, 1, HIDDEN)
    b2r = b2.reshape(NUM_EXPERTS, 1, D_MODEL)

    p0, p1, w0, w1g, be = _route(xf, wgh, bg2)
    p0f = p0.reshape(SEQ)
    p1f = p1.reshape(SEQ)
    w0f = w0.reshape(SEQ)
    w1f = w1g.reshape(SEQ)
    bef = be.reshape(40)[:NUM_FFN_BLOCKS]

    xs = _dispatch(xf, p0f, p1f)
    outs = _ffn(bef, xs, w1, b1r, w2, b2r)
    y = _combine(outs, p0f, p1f, w0f, w1f)
    return y.reshape(b, s, d)


# speedup vs baseline: 1.1593x; 1.1593x over previous
import jax
import jax.numpy as jnp
from jax.experimental import pallas as pl
from jax.experimental.pallas import tpu as pltpu

D_MODEL = 768
NUM_EXPERTS = 8
HIDDEN = 1536
SEQ = 2048
TOKEN_BLOCK = 256
NUM_BLOCKS = SEQ // TOKEN_BLOCK


def _moe_block_kernel(x_ref, wg_ref, bg_ref, w1_ref, b1_ref,
                      w2_ref, b2_ref, o_ref):
    xh = x_ref[...]
    logits = (
        jnp.dot(xh, wg_ref[...], preferred_element_type=jnp.float32)
        + bg_ref[...]
    )
    lane = jax.lax.broadcasted_iota(jnp.int32, logits.shape, 1)
    m1 = jnp.max(logits, axis=1, keepdims=True)
    am1 = jnp.min(jnp.where(logits == m1, lane, NUM_EXPERTS), axis=1,
                  keepdims=True)
    masked = jnp.where(lane == am1, -jnp.inf, logits)
    m2 = jnp.max(masked, axis=1, keepdims=True)
    am2 = jnp.min(jnp.where(masked == m2, lane, NUM_EXPERTS), axis=1,
                  keepdims=True)
    p1 = 1.0 / (1.0 + jnp.exp(m2 - m1))
    p2 = 1.0 - p1

    acc = jnp.zeros((TOKEN_BLOCK, D_MODEL), jnp.float32)
    for j in range(NUM_EXPERTS):
        wj = jnp.where(am1 == j, p1, jnp.where(am2 == j, p2, 0.0))
        h = jnp.dot(xh, w1_ref[j], preferred_element_type=jnp.float32)
        h = (h + b1_ref[j]).astype(jnp.bfloat16)
        h = h * jax.nn.sigmoid(h)
        out = jnp.dot(h, w2_ref[j], preferred_element_type=jnp.float32)
        out = out + b2_ref[j]
        acc = acc + wj * out
    o_ref[...] = acc


def kernel(x, Wg, bg, W1, b1, W2, b2):
    b, s, d = x.shape
    xf = x.reshape(s, d).astype(jnp.bfloat16)
    wgh = Wg.astype(jnp.bfloat16)
    w1 = W1.astype(jnp.bfloat16)
    w2 = W2.astype(jnp.bfloat16)
    bg2 = bg.reshape(1, NUM_EXPERTS)
    b1r = b1.reshape(NUM_EXPERTS, 1, HIDDEN)
    b2r = b2.reshape(NUM_EXPERTS, 1, D_MODEL)

    def const3(i):
        return (0, 0, 0)

    def const2(i):
        return (0, 0)

    y = pl.pallas_call(
        _moe_block_kernel,
        grid=(NUM_BLOCKS,),
        in_specs=[
            pl.BlockSpec((TOKEN_BLOCK, D_MODEL), lambda i: (i, 0)),
            pl.BlockSpec((D_MODEL, NUM_EXPERTS), const2),
            pl.BlockSpec((1, NUM_EXPERTS), const2),
            pl.BlockSpec((NUM_EXPERTS, D_MODEL, HIDDEN), const3),
            pl.BlockSpec((NUM_EXPERTS, 1, HIDDEN), const3),
            pl.BlockSpec((NUM_EXPERTS, HIDDEN, D_MODEL), const3),
            pl.BlockSpec((NUM_EXPERTS, 1, D_MODEL), const3),
        ],
        out_specs=pl.BlockSpec((TOKEN_BLOCK, D_MODEL), lambda i: (i, 0)),
        out_shape=jax.ShapeDtypeStruct((s, d), jnp.float32),
        compiler_params=pltpu.CompilerParams(
            dimension_semantics=("parallel",),
        ),
    )(xf, wgh, bg2, w1, b1r, w2, b2r)
    return y.reshape(b, s, d)


# speedup vs baseline: 1.2253x; 1.0569x over previous
import jax
import jax.numpy as jnp
from jax.experimental import pallas as pl
from jax.experimental.pallas import tpu as pltpu

D_MODEL = 768
NUM_EXPERTS = 8
HIDDEN = 1536
SEQ = 2048
TOKEN_BLOCK = 512
NUM_BLOCKS = SEQ // TOKEN_BLOCK


def _moe_block_kernel(x_ref, wg_ref, bg_ref, w1_ref, b1_ref,
                      w2_ref, b2_ref, o_ref):
    xb = x_ref[...]
    xh = xb.astype(jnp.bfloat16)
    logits = (
        jnp.dot(xh, wg_ref[...], preferred_element_type=jnp.float32)
        + bg_ref[...]
    )
    lane = jax.lax.broadcasted_iota(jnp.int32, logits.shape, 1)
    m1 = jnp.max(logits, axis=1, keepdims=True)
    am1 = jnp.min(jnp.where(logits == m1, lane, NUM_EXPERTS), axis=1,
                  keepdims=True)
    masked = jnp.where(lane == am1, -jnp.inf, logits)
    m2 = jnp.max(masked, axis=1, keepdims=True)
    am2 = jnp.min(jnp.where(masked == m2, lane, NUM_EXPERTS), axis=1,
                  keepdims=True)
    p1 = 1.0 / (1.0 + jnp.exp(m2 - m1))
    p2 = 1.0 - p1

    wmat = jnp.where(lane == am1, p1, jnp.where(lane == am2, p2, 0.0))
    acc = jnp.dot(wmat.astype(jnp.bfloat16),
                  b2_ref[...].astype(jnp.bfloat16),
                  preferred_element_type=jnp.float32)
    for j in range(NUM_EXPERTS):
        wj = wmat[:, j:j + 1]
        h = jnp.dot(xh, w1_ref[j], preferred_element_type=jnp.float32)
        h = h + b1_ref[j]
        h = h * jax.nn.sigmoid(h)
        out = jnp.dot(h.astype(jnp.bfloat16), w2_ref[j],
                      preferred_element_type=jnp.float32)
        acc = acc + wj * out
    o_ref[...] = acc


def kernel(x, Wg, bg, W1, b1, W2, b2):
    b, s, d = x.shape
    xf = x.reshape(s, d)
    wgh = Wg.astype(jnp.bfloat16)
    w1 = W1.astype(jnp.bfloat16)
    w2 = W2.astype(jnp.bfloat16)
    bg2 = bg.reshape(1, NUM_EXPERTS)
    b1r = b1.reshape(NUM_EXPERTS, 1, HIDDEN)
    b2r = b2.reshape(NUM_EXPERTS, D_MODEL)

    def const3(i):
        return (0, 0, 0)

    def const2(i):
        return (0, 0)

    y = pl.pallas_call(
        _moe_block_kernel,
        grid=(NUM_BLOCKS,),
        in_specs=[
            pl.BlockSpec((TOKEN_BLOCK, D_MODEL), lambda i: (i, 0)),
            pl.BlockSpec((D_MODEL, NUM_EXPERTS), const2),
            pl.BlockSpec((1, NUM_EXPERTS), const2),
            pl.BlockSpec((NUM_EXPERTS, D_MODEL, HIDDEN), const3),
            pl.BlockSpec((NUM_EXPERTS, 1, HIDDEN), const3),
            pl.BlockSpec((NUM_EXPERTS, HIDDEN, D_MODEL), const3),
            pl.BlockSpec((NUM_EXPERTS, D_MODEL), const2),
        ],
        out_specs=pl.BlockSpec((TOKEN_BLOCK, D_MODEL), lambda i: (i, 0)),
        out_shape=jax.ShapeDtypeStruct((s, d), jnp.float32),
        compiler_params=pltpu.CompilerParams(
            dimension_semantics=("parallel",),
        ),
    )(xf, wgh, bg2, w1, b1r, w2, b2r)
    return y.reshape(b, s, d)
